# Initial kernel scaffold; baseline (speedup 1.0000x reference)
import jax
import jax.numpy as jnp
from jax.experimental import pallas as pl


def kernel(embeds, edge_index, adj_values, Wq, Wk, Wv):
    raise NotImplementedError("write your pallas kernel here")


# speedup vs baseline: 14.2553x; 14.2553x over previous
import functools

import jax
import jax.numpy as jnp
from jax import lax
from jax.experimental import pallas as pl
from jax.experimental.pallas import tpu as pltpu
from jax.experimental.pallas import tpu_sc as plsc

_N = 10000
_E = 320000
_D = 128
_H = 4
_NC = 2
_NS = 16
_NW = _NC * _NS

_BPW = 80
_NBLK = _NW * _BPW
_EPAD = _NBLK * 128
_SPAN = 4
_NSPAN = _BPW // _SPAN

_NBANK = 6
_BANKQ = 1672
_DUMMY = 1672
_ACCROWS = 1680
_ZS = 104

_INV_SQRT_DH = 1.0 / (32.0 ** 0.5)

_mesh = plsc.VectorSubcoreMesh(core_axis_name="c", subcore_axis_name="s",
                               num_cores=_NC, num_subcores=_NS)
_sc_params = pltpu.CompilerParams(needs_layout_passes=False)


def _zero_rows(buf, nrows, width):
    z = jnp.zeros((16,), jnp.float32)

    def zrow(e, carry):
        for j in range(width // 16):
            buf[e, pl.ds(16 * j, 16)] = z
        return carry

    lax.fori_loop(0, nrows, zrow, 0)


def _bank_geom(b):
    lo = b * _BANKQ
    size = jnp.where(b == _NBANK - 1, _N - (_NBANK - 1) * _BANKQ, _BANKQ)
    pref = lax.rem(lo + _DUMMY, _N)
    return lo, size, pref


def _zero_bank(acc, zsrc, sid):
    base = sid * _ZS
    nch = jnp.where(sid == _NS - 1, (_ACCROWS - 15 * _ZS) // 8, _ZS // 8)

    def zc(r, carry):
        pltpu.sync_copy(zsrc, acc.at[pl.ds(base + 8 * r, 8)])
        return carry

    lax.fori_loop(0, nch, zc, 0)


def _copy_bank(acc, out_hbm, cid, sid, lo, size):
    sbase = sid * _ZS
    nch = jnp.where(sid == _NS - 1, (size - 15 * _ZS) // 8, _ZS // 8)

    def cc(r, carry):
        off = sbase + 8 * r
        pltpu.sync_copy(acc.at[pl.ds(off, 8)],
                        out_hbm.at[cid, pl.ds(lo + off, 8)])
        return carry

    lax.fori_loop(0, nch, cc, 0)


def _compact_span(ridx_all, cidx_all, vals_all, rsel, csel, vsel, sb, lo, hi, pref):
    pos = 0
    for g in range(8 * _SPAN):
        blkr = _SPAN * sb + (g // 8)
        off = 16 * (g % 8)
        rv = ridx_all[blkr, pl.ds(off, 16)]
        cv = cidx_all[blkr, pl.ds(off, 16)]
        m = (rv >= lo) & (rv < hi)
        plsc.store_compressed(rsel.at[pl.ds(pos, 16)], rv, mask=m)
        plsc.store_compressed(csel.at[pl.ds(pos, 16)], cv, mask=m)
        if vals_all is not None:
            vv = vals_all[blkr, pl.ds(off, 16)]
            plsc.store_compressed(vsel.at[pl.ds(pos, 16)], vv, mask=m)
        pos = pos + plsc.all_reduce_population_count(m)[0]
    rsel[pl.ds(pos, 16)] = jnp.full((16,), pref, jnp.int32)
    csel[pl.ds(pos, 16)] = jnp.zeros((16,), jnp.int32)
    if vals_all is not None:
        vsel[pl.ds(pos, 16)] = jnp.zeros((16,), jnp.float32)
    return pos


def _build_seg_targets(rsel, rsel2d, wbase, lo, size):
    for s in range(8):
        rv = rsel[pl.ds(wbase + 16 * s, 16)]
        loc = rv - lo
        ok = (loc >= 0) & (loc < size)
        rsel2d[s] = jnp.where(ok, loc, _DUMMY)


def _spmm_sc(x, rows2d, cols2d, vals2d):
    @functools.partial(
        pl.kernel,
        out_type=jax.ShapeDtypeStruct((_NC, _N, _D), jnp.float32),
        mesh=_mesh,
        compiler_params=_sc_params,
        scratch_types=[
            pltpu.VMEM((_BPW, 128), jnp.int32),
            pltpu.VMEM((_BPW, 128), jnp.int32),
            pltpu.VMEM((_BPW, 128), jnp.float32),
            pltpu.VMEM((528,), jnp.int32),
            pltpu.VMEM((528,), jnp.int32),
            pltpu.VMEM((528,), jnp.float32),
            pltpu.VMEM((8, 16), jnp.int32),
            pltpu.VMEM((128, _D), jnp.float32),
            pltpu.VMEM((8, _D), jnp.float32),
            pltpu.VMEM_SHARED((_ACCROWS, _D), jnp.float32),
            pltpu.SemaphoreType.DMA,
        ],
    )
    def run(x_hbm, rows_hbm, cols_hbm, vals_hbm, out_hbm,
            ridx_all, cidx_all, vals_all, rsel, csel, vsel, rsel2d,
            gbuf, zbuf, acc, sem):
        cid = lax.axis_index("c")
        sid = lax.axis_index("s")
        wid = cid * _NS + sid

        _zero_rows(zbuf, 8, _D)
        pltpu.sync_copy(rows_hbm.at[pl.ds(wid * _BPW, _BPW)], ridx_all)
        pltpu.sync_copy(cols_hbm.at[pl.ds(wid * _BPW, _BPW)], cidx_all)
        pltpu.sync_copy(vals_hbm.at[pl.ds(wid * _BPW, _BPW)], vals_all)

        def bank_body(b, carry):
            lo, size, pref = _bank_geom(b)
            hi = lo + size
            _zero_bank(acc, zbuf, sid)
            plsc.subcore_barrier()

            def span_body(sb, c2):
                pos = _compact_span(ridx_all, cidx_all, vals_all,
                                    rsel, csel, vsel, sb, lo, hi, pref)
                nwave = (pos + 127) // 128

                def wave(w, c3):
                    wbase = 128 * w
                    cnt = jnp.minimum(pos - wbase, 128)
                    nseg = (cnt + 15) // 16

                    def fire(s, c4):
                        pltpu.async_copy(
                            x_hbm.at[csel.at[pl.ds(wbase + 16 * s, 16)]],
                            gbuf.at[pl.ds(16 * s, 16)], sem)
                        return c4

                    lax.fori_loop(0, nseg, fire, 0)

                    def drain(s, c4):
                        pltpu.make_async_copy(
                            x_hbm.at[csel.at[pl.ds(wbase + 16 * s, 16)]],
                            gbuf.at[pl.ds(16 * s, 16)], sem).wait()
                        return c4

                    lax.fori_loop(0, nseg, drain, 0)

                    def scale(i, c4):
                        vv = vsel[pl.ds(wbase + 16 * i, 16)]
                        for ee in range(16):
                            e = 16 * i + ee
                            bs = jnp.full((16,), vv[ee], jnp.float32)
                            for jj in range(_D // 16):
                                gbuf[e, pl.ds(16 * jj, 16)] = (
                                    gbuf[e, pl.ds(16 * jj, 16)] * bs)
                        return c4

                    lax.fori_loop(0, nseg, scale, 0)
                    _build_seg_targets(rsel, rsel2d, wbase, lo, size)

                    def sfire(s, c4):
                        pltpu.async_copy(gbuf.at[pl.ds(16 * s, 16)],
                                         acc.at[rsel2d.at[s]], sem, add=True)
                        return c4

                    lax.fori_loop(0, nseg, sfire, 0)

                    def sdrain(s, c4):
                        pltpu.make_async_copy(gbuf.at[pl.ds(16 * s, 16)],
                                              acc.at[rsel2d.at[s]], sem).wait()
                        return c4

                    lax.fori_loop(0, nseg, sdrain, 0)
                    return c3

                lax.fori_loop(0, nwave, wave, 0)
                return c2

            lax.fori_loop(0, _NSPAN, span_body, 0)
            plsc.subcore_barrier()
            _copy_bank(acc, out_hbm, cid, sid, lo, size)
            plsc.subcore_barrier()
            return carry

        lax.fori_loop(0, _NBANK, bank_body, 0)

    return run(x, rows2d, cols2d, vals2d)


def _gt_sc(q, k, v, rows2d, cols2d):
    @functools.partial(
        pl.kernel,
        out_type=(
            jax.ShapeDtypeStruct((_NC, _N, _D), jnp.float32),
            jax.ShapeDtypeStruct((_NC, _N, _D), jnp.float32),
        ),
        mesh=_mesh,
        compiler_params=_sc_params,
        scratch_types=[
            pltpu.VMEM((_BPW, 128), jnp.int32),
            pltpu.VMEM((_BPW, 128), jnp.int32),
            pltpu.VMEM((528,), jnp.int32),
            pltpu.VMEM((528,), jnp.int32),
            pltpu.VMEM((8, 16), jnp.int32),
            pltpu.VMEM((128, _D), jnp.float32),
            pltpu.VMEM((128, _D), jnp.float32),
            pltpu.VMEM((128, _D), jnp.float32),
            pltpu.VMEM((2048,), jnp.float32),
            pltpu.VMEM((2048,), jnp.float32),
            pltpu.VMEM((128, _D), jnp.float32),
            pltpu.VMEM((8, _D), jnp.float32),
            pltpu.VMEM_SHARED((_ACCROWS, _D), jnp.float32),
            pltpu.VMEM_SHARED((_ACCROWS, _D), jnp.float32),
            pltpu.SemaphoreType.DMA,
        ],
    )
    def run(q_hbm, k_hbm, v_hbm, rows_hbm, cols_hbm, onum_hbm, oden_hbm,
            ridx_all, cidx_all, rsel, csel, rsel2d, qbuf, kbuf, vbuf,
            araw, aexp, dstage, zbuf, nacc, dacc, sem):
        cid = lax.axis_index("c")
        sid = lax.axis_index("s")
        wid = cid * _NS + sid
        lane = lax.iota(jnp.int32, 16)

        _zero_rows(zbuf, 8, _D)
        _zero_rows(dstage, 128, _D)
        pltpu.sync_copy(rows_hbm.at[pl.ds(wid * _BPW, _BPW)], ridx_all)
        pltpu.sync_copy(cols_hbm.at[pl.ds(wid * _BPW, _BPW)], cidx_all)

        def bank_body(b, carry):
            lo, size, pref = _bank_geom(b)
            hi = lo + size
            _zero_bank(nacc, zbuf, sid)
            _zero_bank(dacc, zbuf, sid)
            plsc.subcore_barrier()

            def span_body(sb, c2):
                pos = _compact_span(ridx_all, cidx_all, None,
                                    rsel, csel, None, sb, lo, hi, pref)
                nwave = (pos + 127) // 128

                def wave(w, c3):
                    wbase = 128 * w
                    cnt = jnp.minimum(pos - wbase, 128)
                    nseg = (cnt + 15) // 16

                    def fire(s, c4):
                        o = wbase + 16 * s
                        pltpu.async_copy(q_hbm.at[rsel.at[pl.ds(o, 16)]],
                                         qbuf.at[pl.ds(16 * s, 16)], sem)
                        pltpu.async_copy(k_hbm.at[csel.at[pl.ds(o, 16)]],
                                         kbuf.at[pl.ds(16 * s, 16)], sem)
                        pltpu.async_copy(v_hbm.at[csel.at[pl.ds(o, 16)]],
                                         vbuf.at[pl.ds(16 * s, 16)], sem)
                        return c4

                    lax.fori_loop(0, nseg, fire, 0)

                    def drain(s, c4):
                        o = wbase + 16 * s
                        pltpu.make_async_copy(
                            q_hbm.at[rsel.at[pl.ds(o, 16)]],
                            qbuf.at[pl.ds(16 * s, 16)], sem).wait()
                        pltpu.make_async_copy(
                            k_hbm.at[csel.at[pl.ds(o, 16)]],
                            kbuf.at[pl.ds(16 * s, 16)], sem).wait()
                        pltpu.make_async_copy(
                            v_hbm.at[csel.at[pl.ds(o, 16)]],
                            vbuf.at[pl.ds(16 * s, 16)], sem).wait()
                        return c4

                    lax.fori_loop(0, nseg, drain, 0)

                    def dots(e, c4):
                        t = jnp.zeros((16,), jnp.float32)
                        for h in range(_H):
                            m = (qbuf[e, pl.ds(32 * h, 16)]
                                 * kbuf[e, pl.ds(32 * h, 16)]
                                 + qbuf[e, pl.ds(32 * h + 16, 16)]
                                 * kbuf[e, pl.ds(32 * h + 16, 16)])
                            t = jnp.where(
                                lane == h,
                                jnp.full((16,), jnp.sum(m), jnp.float32), t)
                        araw[pl.ds(16 * e, 16)] = t
                        return c4

                    lax.fori_loop(0, cnt, dots, 0)

                    def vexp(i, c4):
                        tv = araw[pl.ds(16 * i, 16)] * _INV_SQRT_DH
                        tv = jnp.clip(tv, -10.0, 10.0)
                        aexp[pl.ds(16 * i, 16)] = jnp.exp(tv)
                        return c4

                    lax.fori_loop(0, cnt, vexp, 0)

                    def apply(e, c4):
                        av = aexp[pl.ds(16 * e, 16)]
                        for h in range(_H):
                            bs = jnp.full((16,), av[h], jnp.float32)
                            dstage[e, pl.ds(32 * h, 16)] = bs
                            dstage[e, pl.ds(32 * h + 16, 16)] = bs
                            vbuf[e, pl.ds(32 * h, 16)] = (
                                vbuf[e, pl.ds(32 * h, 16)] * bs)
                            vbuf[e, pl.ds(32 * h + 16, 16)] = (
                                vbuf[e, pl.ds(32 * h + 16, 16)] * bs)
                        return c4

                    lax.fori_loop(0, cnt, apply, 0)
                    _build_seg_targets(rsel, rsel2d, wbase, lo, size)

                    def sfire(s, c4):
                        pltpu.async_copy(vbuf.at[pl.ds(16 * s, 16)],
                                         nacc.at[rsel2d.at[s]], sem, add=True)
                        pltpu.async_copy(dstage.at[pl.ds(16 * s, 16)],
                                         dacc.at[rsel2d.at[s]], sem, add=True)
                        return c4

                    lax.fori_loop(0, nseg, sfire, 0)

                    def sdrain(s, c4):
                        pltpu.make_async_copy(vbuf.at[pl.ds(16 * s, 16)],
                                              nacc.at[rsel2d.at[s]], sem).wait()
                        pltpu.make_async_copy(dstage.at[pl.ds(16 * s, 16)],
                                              dacc.at[rsel2d.at[s]], sem).wait()
                        return c4

                    lax.fori_loop(0, nseg, sdrain, 0)
                    return c3

                lax.fori_loop(0, nwave, wave, 0)
                return c2

            lax.fori_loop(0, _NSPAN, span_body, 0)
            plsc.subcore_barrier()
            _copy_bank(nacc, onum_hbm, cid, sid, lo, size)
            _copy_bank(dacc, oden_hbm, cid, sid, lo, size)
            plsc.subcore_barrier()
            return carry

        lax.fori_loop(0, _NBANK, bank_body, 0)

    return run(q, k, v, rows2d, cols2d)


_ROWS_PER_TC_BLK = 1000
_TC_GRID = _N // _ROWS_PER_TC_BLK


def _qkv_tc(parts, run_in, Wq, Wk, Wv):
    def body(p_ref, run_ref, wq_ref, wk_ref, wv_ref, q_ref, k_ref, v_ref, nrun_ref):
        x = p_ref[0] + p_ref[1]
        q_ref[...] = jnp.dot(x, wq_ref[...], preferred_element_type=jnp.float32)
        k_ref[...] = jnp.dot(x, wk_ref[...], preferred_element_type=jnp.float32)
        v_ref[...] = jnp.dot(x, wv_ref[...], preferred_element_type=jnp.float32)
        nrun_ref[...] = run_ref[...] + x

    nd = jax.ShapeDtypeStruct((_N, _D), jnp.float32)
    return pl.pallas_call(
        body,
        grid=(_TC_GRID,),
        in_specs=[
            pl.BlockSpec((_NC, _ROWS_PER_TC_BLK, _D), lambda i: (0, i, 0)),
            pl.BlockSpec((_ROWS_PER_TC_BLK, _D), lambda i: (i, 0)),
            pl.BlockSpec((_D, _D), lambda i: (0, 0)),
            pl.BlockSpec((_D, _D), lambda i: (0, 0)),
            pl.BlockSpec((_D, _D), lambda i: (0, 0)),
        ],
        out_specs=[pl.BlockSpec((_ROWS_PER_TC_BLK, _D), lambda i: (i, 0))] * 4,
        out_shape=[nd, nd, nd, nd],
    )(parts, run_in, Wq, Wk, Wv)


def _attn_out(n_ref, d_ref):
    nsum = n_ref[0] + n_ref[1]
    dsum = d_ref[0] + d_ref[1]
    return nsum / (dsum + 1e-8)


def _epi_tc(numer_parts, denom_parts, run_in):
    def body(n_ref, d_ref, run_ref, x_ref, nrun_ref):
        x = _attn_out(n_ref, d_ref)
        x_ref[...] = x
        nrun_ref[...] = run_ref[...] + x

    nd = jax.ShapeDtypeStruct((_N, _D), jnp.float32)
    return pl.pallas_call(
        body,
        grid=(_TC_GRID,),
        in_specs=[
            pl.BlockSpec((_NC, _ROWS_PER_TC_BLK, _D), lambda i: (0, i, 0)),
            pl.BlockSpec((_NC, _ROWS_PER_TC_BLK, _D), lambda i: (0, i, 0)),
            pl.BlockSpec((_ROWS_PER_TC_BLK, _D), lambda i: (i, 0)),
        ],
        out_specs=[pl.BlockSpec((_ROWS_PER_TC_BLK, _D), lambda i: (i, 0))] * 2,
        out_shape=[nd, nd],
    )(numer_parts, denom_parts, run_in)


def _final_tc(numer_parts, denom_parts, run_in):
    def body(n_ref, d_ref, run_ref, o_ref):
        o_ref[...] = run_ref[...] + _attn_out(n_ref, d_ref)

    return pl.pallas_call(
        body,
        grid=(_TC_GRID,),
        in_specs=[
            pl.BlockSpec((_NC, _ROWS_PER_TC_BLK, _D), lambda i: (0, i, 0)),
            pl.BlockSpec((_NC, _ROWS_PER_TC_BLK, _D), lambda i: (0, i, 0)),
            pl.BlockSpec((_ROWS_PER_TC_BLK, _D), lambda i: (i, 0)),
        ],
        out_specs=pl.BlockSpec((_ROWS_PER_TC_BLK, _D), lambda i: (i, 0)),
        out_shape=jax.ShapeDtypeStruct((_N, _D), jnp.float32),
    )(numer_parts, denom_parts, run_in)


def kernel(embeds, edge_index, adj_values, Wq, Wk, Wv):
    npad = _EPAD - _E
    rows2d = jnp.concatenate(
        [edge_index[0], jnp.full((npad,), _N, jnp.int32)]).reshape(_NBLK, 128)
    cols2d = jnp.concatenate(
        [edge_index[1], jnp.zeros((npad,), jnp.int32)]).reshape(_NBLK, 128)
    vals2d = jnp.concatenate(
        [adj_values, jnp.zeros((npad,), jnp.float32)]).reshape(_NBLK, 128)

    p1 = _spmm_sc(embeds, rows2d, cols2d, vals2d)
    q1, k1, v1, run1 = _qkv_tc(p1, embeds, Wq, Wk, Wv)
    n1, d1 = _gt_sc(q1, k1, v1, rows2d, cols2d)
    x2, run2 = _epi_tc(n1, d1, run1)
    p3 = _spmm_sc(x2, rows2d, cols2d, vals2d)
    q3, k3, v3, run3 = _qkv_tc(p3, run2, Wq, Wk, Wv)
    n3, d3 = _gt_sc(q3, k3, v3, rows2d, cols2d)
    return _final_tc(n3, d3, run3)


# speedup vs baseline: 18.3949x; 1.2904x over previous
import functools

import jax
import jax.numpy as jnp
from jax import lax
from jax.experimental import pallas as pl
from jax.experimental.pallas import tpu as pltpu
from jax.experimental.pallas import tpu_sc as plsc

_N = 10000
_E = 320000
_D = 128
_H = 4
_NC = 2
_NS = 16
_NW = _NC * _NS

_BPW = 80
_NBLK = _NW * _BPW
_EPAD = _NBLK * 128
_SPAN = 4
_NSPAN = _BPW // _SPAN

_NBANK = 6
_BANKQ = 1672
_DUMMY = 1672
_ACCROWS = 1680
_ZS = 104

_INV_SQRT_DH = 1.0 / (32.0 ** 0.5)

_mesh = plsc.VectorSubcoreMesh(core_axis_name="c", subcore_axis_name="s",
                               num_cores=_NC, num_subcores=_NS)
_sc_params = pltpu.CompilerParams(needs_layout_passes=False)


def _zero_rows(buf, nrows, width):
    z = jnp.zeros((16,), jnp.float32)

    def zrow(e, carry):
        for j in range(width // 16):
            buf[e, pl.ds(16 * j, 16)] = z
        return carry

    lax.fori_loop(0, nrows, zrow, 0)


def _bank_geom(b):
    lo = b * _BANKQ
    size = jnp.where(b == _NBANK - 1, _N - (_NBANK - 1) * _BANKQ, _BANKQ)
    pref = lax.rem(lo + _DUMMY, _N)
    return lo, size, pref


def _zero_bank(acc, zsrc, sid):
    base = sid * _ZS
    nch = jnp.where(sid == _NS - 1, (_ACCROWS - 15 * _ZS) // 8, _ZS // 8)

    def zc(r, carry):
        pltpu.sync_copy(zsrc, acc.at[pl.ds(base + 8 * r, 8)])
        return carry

    lax.fori_loop(0, nch, zc, 0)


def _copy_bank(acc, out_hbm, cid, sid, lo, size):
    sbase = sid * _ZS
    nch = jnp.where(sid == _NS - 1, (size - 15 * _ZS) // 8, _ZS // 8)

    def cc(r, carry):
        off = sbase + 8 * r
        pltpu.sync_copy(acc.at[pl.ds(off, 8)],
                        out_hbm.at[cid, pl.ds(lo + off, 8)])
        return carry

    lax.fori_loop(0, nch, cc, 0)


def _compact_span(ridx_all, cidx_all, vals_all, rsel, csel, vsel, sb, lo, hi, pref):
    pos = 0
    for g in range(8 * _SPAN):
        blkr = _SPAN * sb + (g // 8)
        off = 16 * (g % 8)
        rv = ridx_all[blkr, pl.ds(off, 16)]
        cv = cidx_all[blkr, pl.ds(off, 16)]
        m = (rv >= lo) & (rv < hi)
        plsc.store_compressed(rsel.at[pl.ds(pos, 16)], rv, mask=m)
        plsc.store_compressed(csel.at[pl.ds(pos, 16)], cv, mask=m)
        if vals_all is not None:
            vv = vals_all[blkr, pl.ds(off, 16)]
            plsc.store_compressed(vsel.at[pl.ds(pos, 16)], vv, mask=m)
        pos = pos + plsc.all_reduce_population_count(m)[0]
    rsel[pl.ds(pos, 16)] = jnp.full((16,), pref, jnp.int32)
    csel[pl.ds(pos, 16)] = jnp.zeros((16,), jnp.int32)
    if vals_all is not None:
        vsel[pl.ds(pos, 16)] = jnp.zeros((16,), jnp.float32)
    return pos


def _build_seg_targets(rsel, rsel2d, wbase, lo, size):
    for s in range(8):
        rv = rsel[pl.ds(wbase + 16 * s, 16)]
        loc = rv - lo
        ok = (loc >= 0) & (loc < size)
        rsel2d[s] = jnp.where(ok, loc, _DUMMY)


_FACC = 10008
_FZ = 624
_FC = 624


def _zero_full(acc, zsrc, sid):
    base = sid * _FZ
    nch = jnp.where(sid == _NS - 1, (_FACC - 15 * _FZ) // 8, _FZ // 8)

    def zc(r, carry):
        pltpu.sync_copy(zsrc, acc.at[pl.ds(base + 8 * r, 8)])
        return carry

    lax.fori_loop(0, nch, zc, 0)


def _copy_full(acc, out_hbm, cid, sid):
    base = sid * _FC
    nch = jnp.where(sid == _NS - 1, (_N - 15 * _FC) // 8, _FC // 8)

    def cc(r, carry):
        off = base + 8 * r
        pltpu.sync_copy(acc.at[pl.ds(off, 8)],
                        out_hbm.at[cid, pl.ds(off, 8)])
        return carry

    lax.fori_loop(0, nch, cc, 0)


def _spmm_sc(x, rows2d, cols2d, vals2d):
    @functools.partial(
        pl.kernel,
        out_type=jax.ShapeDtypeStruct((_NC, _N, _D), jnp.float32),
        mesh=_mesh,
        compiler_params=_sc_params,
        scratch_types=[
            pltpu.VMEM((_BPW, 128), jnp.int32),
            pltpu.VMEM((_BPW, 128), jnp.int32),
            pltpu.VMEM((_BPW, 128), jnp.float32),
            pltpu.VMEM((128, _D), jnp.float32),
            pltpu.VMEM((8, _D), jnp.float32),
            pltpu.VMEM_SHARED((_FACC, _D), jnp.float32),
            pltpu.SemaphoreType.DMA,
        ],
    )
    def run(x_hbm, rows_hbm, cols_hbm, vals_hbm, out_hbm,
            ridx_all, cidx_all, vals_all, gbuf, zbuf, acc, sem):
        cid = lax.axis_index("c")
        sid = lax.axis_index("s")
        wid = cid * _NS + sid

        _zero_rows(zbuf, 8, _D)
        pltpu.sync_copy(rows_hbm.at[pl.ds(wid * _BPW, _BPW)], ridx_all)
        pltpu.sync_copy(cols_hbm.at[pl.ds(wid * _BPW, _BPW)], cidx_all)
        pltpu.sync_copy(vals_hbm.at[pl.ds(wid * _BPW, _BPW)], vals_all)
        _zero_full(acc, zbuf, sid)
        plsc.subcore_barrier()

        def blk(b, carry):
            pltpu.async_copy(x_hbm.at[cidx_all.at[b]], gbuf, sem).wait()

            def scale(i, c4):
                vv = vals_all[b, pl.ds(16 * i, 16)]
                for ee in range(16):
                    e = 16 * i + ee
                    bs = jnp.full((16,), vv[ee], jnp.float32)
                    for jj in range(_D // 16):
                        gbuf[e, pl.ds(16 * jj, 16)] = (
                            gbuf[e, pl.ds(16 * jj, 16)] * bs)
                return c4

            lax.fori_loop(0, 8, scale, 0)
            pltpu.async_copy(gbuf, acc.at[ridx_all.at[b]], sem, add=True).wait()
            return carry

        lax.fori_loop(0, _BPW, blk, 0)
        plsc.subcore_barrier()
        _copy_full(acc, out_hbm, cid, sid)

    return run(x, rows2d, cols2d, vals2d)


def _gt_sc(q, k, v, rows2d, cols2d):
    @functools.partial(
        pl.kernel,
        out_type=(
            jax.ShapeDtypeStruct((_NC, _N, _D), jnp.float32),
            jax.ShapeDtypeStruct((_NC, _N, _D), jnp.float32),
        ),
        mesh=_mesh,
        compiler_params=_sc_params,
        scratch_types=[
            pltpu.VMEM((_BPW, 128), jnp.int32),
            pltpu.VMEM((_BPW, 128), jnp.int32),
            pltpu.VMEM((528,), jnp.int32),
            pltpu.VMEM((528,), jnp.int32),
            pltpu.VMEM((8, 16), jnp.int32),
            pltpu.VMEM((128, _D), jnp.float32),
            pltpu.VMEM((128, _D), jnp.float32),
            pltpu.VMEM((128, _D), jnp.float32),
            pltpu.VMEM((2048,), jnp.float32),
            pltpu.VMEM((2048,), jnp.float32),
            pltpu.VMEM((128, _D), jnp.float32),
            pltpu.VMEM((8, _D), jnp.float32),
            pltpu.VMEM_SHARED((_ACCROWS, _D), jnp.float32),
            pltpu.VMEM_SHARED((_ACCROWS, _D), jnp.float32),
            pltpu.SemaphoreType.DMA,
        ],
    )
    def run(q_hbm, k_hbm, v_hbm, rows_hbm, cols_hbm, onum_hbm, oden_hbm,
            ridx_all, cidx_all, rsel, csel, rsel2d, qbuf, kbuf, vbuf,
            araw, aexp, dstage, zbuf, nacc, dacc, sem):
        cid = lax.axis_index("c")
        sid = lax.axis_index("s")
        wid = cid * _NS + sid
        lane = lax.iota(jnp.int32, 16)

        _zero_rows(zbuf, 8, _D)
        _zero_rows(dstage, 128, _D)
        pltpu.sync_copy(rows_hbm.at[pl.ds(wid * _BPW, _BPW)], ridx_all)
        pltpu.sync_copy(cols_hbm.at[pl.ds(wid * _BPW, _BPW)], cidx_all)

        def bank_body(b, carry):
            lo, size, pref = _bank_geom(b)
            hi = lo + size
            _zero_bank(nacc, zbuf, sid)
            _zero_bank(dacc, zbuf, sid)
            plsc.subcore_barrier()

            def span_body(sb, c2):
                pos = _compact_span(ridx_all, cidx_all, None,
                                    rsel, csel, None, sb, lo, hi, pref)
                nwave = (pos + 127) // 128

                def wave(w, c3):
                    wbase = 128 * w
                    cnt = jnp.minimum(pos - wbase, 128)
                    nseg = (cnt + 15) // 16

                    def fire(s, c4):
                        o = wbase + 16 * s
                        pltpu.async_copy(q_hbm.at[rsel.at[pl.ds(o, 16)]],
                                         qbuf.at[pl.ds(16 * s, 16)], sem)
                        pltpu.async_copy(k_hbm.at[csel.at[pl.ds(o, 16)]],
                                         kbuf.at[pl.ds(16 * s, 16)], sem)
                        pltpu.async_copy(v_hbm.at[csel.at[pl.ds(o, 16)]],
                                         vbuf.at[pl.ds(16 * s, 16)], sem)
                        return c4

                    lax.fori_loop(0, nseg, fire, 0)

                    def drain(s, c4):
                        o = wbase + 16 * s
                        pltpu.make_async_copy(
                            q_hbm.at[rsel.at[pl.ds(o, 16)]],
                            qbuf.at[pl.ds(16 * s, 16)], sem).wait()
                        pltpu.make_async_copy(
                            k_hbm.at[csel.at[pl.ds(o, 16)]],
                            kbuf.at[pl.ds(16 * s, 16)], sem).wait()
                        pltpu.make_async_copy(
                            v_hbm.at[csel.at[pl.ds(o, 16)]],
                            vbuf.at[pl.ds(16 * s, 16)], sem).wait()
                        return c4

                    lax.fori_loop(0, nseg, drain, 0)

                    def dots(e, c4):
                        t = jnp.zeros((16,), jnp.float32)
                        for h in range(_H):
                            m = (qbuf[e, pl.ds(32 * h, 16)]
                                 * kbuf[e, pl.ds(32 * h, 16)]
                                 + qbuf[e, pl.ds(32 * h + 16, 16)]
                                 * kbuf[e, pl.ds(32 * h + 16, 16)])
                            t = jnp.where(
                                lane == h,
                                jnp.full((16,), jnp.sum(m), jnp.float32), t)
                        araw[pl.ds(16 * e, 16)] = t
                        return c4

                    lax.fori_loop(0, cnt, dots, 0)

                    def vexp(i, c4):
                        tv = araw[pl.ds(16 * i, 16)] * _INV_SQRT_DH
                        tv = jnp.clip(tv, -10.0, 10.0)
                        aexp[pl.ds(16 * i, 16)] = jnp.exp(tv)
                        return c4

                    lax.fori_loop(0, cnt, vexp, 0)

                    def apply(e, c4):
                        av = aexp[pl.ds(16 * e, 16)]
                        for h in range(_H):
                            bs = jnp.full((16,), av[h], jnp.float32)
                            dstage[e, pl.ds(32 * h, 16)] = bs
                            dstage[e, pl.ds(32 * h + 16, 16)] = bs
                            vbuf[e, pl.ds(32 * h, 16)] = (
                                vbuf[e, pl.ds(32 * h, 16)] * bs)
                            vbuf[e, pl.ds(32 * h + 16, 16)] = (
                                vbuf[e, pl.ds(32 * h + 16, 16)] * bs)
                        return c4

                    lax.fori_loop(0, cnt, apply, 0)
                    _build_seg_targets(rsel, rsel2d, wbase, lo, size)

                    def sfire(s, c4):
                        pltpu.async_copy(vbuf.at[pl.ds(16 * s, 16)],
                                         nacc.at[rsel2d.at[s]], sem, add=True)
                        pltpu.async_copy(dstage.at[pl.ds(16 * s, 16)],
                                         dacc.at[rsel2d.at[s]], sem, add=True)
                        return c4

                    lax.fori_loop(0, nseg, sfire, 0)

                    def sdrain(s, c4):
                        pltpu.make_async_copy(vbuf.at[pl.ds(16 * s, 16)],
                                              nacc.at[rsel2d.at[s]], sem).wait()
                        pltpu.make_async_copy(dstage.at[pl.ds(16 * s, 16)],
                                              dacc.at[rsel2d.at[s]], sem).wait()
                        return c4

                    lax.fori_loop(0, nseg, sdrain, 0)
                    return c3

                lax.fori_loop(0, nwave, wave, 0)
                return c2

            lax.fori_loop(0, _NSPAN, span_body, 0)
            plsc.subcore_barrier()
            _copy_bank(nacc, onum_hbm, cid, sid, lo, size)
            _copy_bank(dacc, oden_hbm, cid, sid, lo, size)
            plsc.subcore_barrier()
            return carry

        lax.fori_loop(0, _NBANK, bank_body, 0)

    return run(q, k, v, rows2d, cols2d)


_ROWS_PER_TC_BLK = 1000
_TC_GRID = _N // _ROWS_PER_TC_BLK


def _qkv_tc(parts, run_in, Wq, Wk, Wv):
    def body(p_ref, run_ref, wq_ref, wk_ref, wv_ref, q_ref, k_ref, v_ref, nrun_ref):
        x = p_ref[0] + p_ref[1]
        q_ref[...] = jnp.dot(x, wq_ref[...], preferred_element_type=jnp.float32)
        k_ref[...] = jnp.dot(x, wk_ref[...], preferred_element_type=jnp.float32)
        v_ref[...] = jnp.dot(x, wv_ref[...], preferred_element_type=jnp.float32)
        nrun_ref[...] = run_ref[...] + x

    nd = jax.ShapeDtypeStruct((_N, _D), jnp.float32)
    return pl.pallas_call(
        body,
        grid=(_TC_GRID,),
        in_specs=[
            pl.BlockSpec((_NC, _ROWS_PER_TC_BLK, _D), lambda i: (0, i, 0)),
            pl.BlockSpec((_ROWS_PER_TC_BLK, _D), lambda i: (i, 0)),
            pl.BlockSpec((_D, _D), lambda i: (0, 0)),
            pl.BlockSpec((_D, _D), lambda i: (0, 0)),
            pl.BlockSpec((_D, _D), lambda i: (0, 0)),
        ],
        out_specs=[pl.BlockSpec((_ROWS_PER_TC_BLK, _D), lambda i: (i, 0))] * 4,
        out_shape=[nd, nd, nd, nd],
    )(parts, run_in, Wq, Wk, Wv)


def _attn_out(n_ref, d_ref):
    nsum = n_ref[0] + n_ref[1]
    dsum = d_ref[0] + d_ref[1]
    return nsum / (dsum + 1e-8)


def _epi_tc(numer_parts, denom_parts, run_in):
    def body(n_ref, d_ref, run_ref, x_ref, nrun_ref):
        x = _attn_out(n_ref, d_ref)
        x_ref[...] = x
        nrun_ref[...] = run_ref[...] + x

    nd = jax.ShapeDtypeStruct((_N, _D), jnp.float32)
    return pl.pallas_call(
        body,
        grid=(_TC_GRID,),
        in_specs=[
            pl.BlockSpec((_NC, _ROWS_PER_TC_BLK, _D), lambda i: (0, i, 0)),
            pl.BlockSpec((_NC, _ROWS_PER_TC_BLK, _D), lambda i: (0, i, 0)),
            pl.BlockSpec((_ROWS_PER_TC_BLK, _D), lambda i: (i, 0)),
        ],
        out_specs=[pl.BlockSpec((_ROWS_PER_TC_BLK, _D), lambda i: (i, 0))] * 2,
        out_shape=[nd, nd],
    )(numer_parts, denom_parts, run_in)


def _final_tc(numer_parts, denom_parts, run_in):
    def body(n_ref, d_ref, run_ref, o_ref):
        o_ref[...] = run_ref[...] + _attn_out(n_ref, d_ref)

    return pl.pallas_call(
        body,
        grid=(_TC_GRID,),
        in_specs=[
            pl.BlockSpec((_NC, _ROWS_PER_TC_BLK, _D), lambda i: (0, i, 0)),
            pl.BlockSpec((_NC, _ROWS_PER_TC_BLK, _D), lambda i: (0, i, 0)),
            pl.BlockSpec((_ROWS_PER_TC_BLK, _D), lambda i: (i, 0)),
        ],
        out_specs=pl.BlockSpec((_ROWS_PER_TC_BLK, _D), lambda i: (i, 0)),
        out_shape=jax.ShapeDtypeStruct((_N, _D), jnp.float32),
    )(numer_parts, denom_parts, run_in)


def kernel(embeds, edge_index, adj_values, Wq, Wk, Wv):
    npad = _EPAD - _E
    rows2d = jnp.concatenate(
        [edge_index[0], jnp.full((npad,), _N, jnp.int32)]).reshape(_NBLK, 128)
    cols2d = jnp.concatenate(
        [edge_index[1], jnp.zeros((npad,), jnp.int32)]).reshape(_NBLK, 128)
    vals2d = jnp.concatenate(
        [adj_values, jnp.zeros((npad,), jnp.float32)]).reshape(_NBLK, 128)

    p1 = _spmm_sc(embeds, rows2d, cols2d, vals2d)
    q1, k1, v1, run1 = _qkv_tc(p1, embeds, Wq, Wk, Wv)
    n1, d1 = _gt_sc(q1, k1, v1, rows2d, cols2d)
    x2, run2 = _epi_tc(n1, d1, run1)
    p3 = _spmm_sc(x2, rows2d, cols2d, vals2d)
    q3, k3, v3, run3 = _qkv_tc(p3, run2, Wq, Wk, Wv)
    n3, d3 = _gt_sc(q3, k3, v3, rows2d, cols2d)
    return _final_tc(n3, d3, run3)
